# baseline (speedup 1.0000x reference)
import functools

import jax
import jax.numpy as jnp
from jax import lax
from jax.experimental import pallas as pl
from jax.experimental.pallas import tpu as pltpu
from jax.experimental.pallas import tpu_sc as plsc

NUM_SIGNALS = 100000
ROW_WIDTH = 70
BATCH = 4096

_info = plsc.get_sparse_core_info()
_NC, _NS = _info.num_cores, _info.num_subcores
_NW = _NC * _NS
_H0 = 49920
_H1 = NUM_SIGNALS - _H0
_NGRP = BATCH // 16
_NTAIL = ROW_WIDTH - 2 * _NW


def _make_gather():
    mesh = plsc.VectorSubcoreMesh(core_axis_name="c", subcore_axis_name="s")

    @functools.partial(
        pl.kernel,
        mesh=mesh,
        out_type=jax.ShapeDtypeStruct((ROW_WIDTH, BATCH), jnp.float32),
        scratch_types=[
            pltpu.VMEM((BATCH,), jnp.int32),
            pltpu.VMEM((_H0,), jnp.float32),
            pltpu.VMEM((_H1,), jnp.float32),
            pltpu.VMEM((BATCH,), jnp.float32),
            pltpu.SemaphoreType.DMA,
            pltpu.SemaphoreType.DMA,
        ],
        compiler_params=pltpu.CompilerParams(
            use_tc_tiling_on_sc=True, needs_layout_passes=False
        ),
    )
    def gather_kernel(idx_hbm, table_hbm, out_hbm, idx_v, a_v, b_v, res_v,
                      sem_a, sem_b):
        wid = lax.axis_index("s") * _NC + lax.axis_index("c")
        pltpu.sync_copy(idx_hbm, idx_v)
        iota = lax.iota(jnp.int32, 16)

        def fetch_a(j):
            pltpu.make_async_copy(
                table_hbm.at[j, pl.ds(0, _H0)], a_v, sem_a).start()

        def fetch_b(j):
            pltpu.make_async_copy(
                table_hbm.at[j, pl.ds(_H0, _H1)], b_v, sem_b).start()

        def wait_a():
            pltpu.make_async_copy(
                table_hbm.at[0, pl.ds(0, _H0)], a_v, sem_a).wait()

        def wait_b():
            pltpu.make_async_copy(
                table_hbm.at[0, pl.ds(_H0, _H1)], b_v, sem_b).wait()

        def pass_a(k, carry):
            iv = idx_v[pl.ds(k * 16, 16)]
            g = plsc.load_gather(a_v, [jnp.minimum(iv, _H0 - 1)])
            plsc.store_scatter(res_v, [iota + k * 16], g, mask=iv < _H0)
            return carry

        def pass_b(k, carry):
            iv = idx_v[pl.ds(k * 16, 16)]
            g = plsc.load_gather(b_v, [jnp.maximum(iv - _H0, 0)])
            plsc.store_scatter(res_v, [iota + k * 16], g, mask=iv >= _H0)
            return carry

        def writeout(j):
            pltpu.sync_copy(res_v, out_hbm.at[j])

        has_third = wid < _NTAIL

        fetch_a(wid)
        fetch_b(wid)
        wait_a()
        lax.fori_loop(0, _NGRP, pass_a, 0)
        fetch_a(wid + _NW)
        wait_b()
        lax.fori_loop(0, _NGRP, pass_b, 0)
        fetch_b(wid + _NW)
        writeout(wid)
        wait_a()
        lax.fori_loop(0, _NGRP, pass_a, 0)

        @pl.when(has_third)
        def _():
            fetch_a(wid + 2 * _NW)

        wait_b()
        lax.fori_loop(0, _NGRP, pass_b, 0)

        @pl.when(has_third)
        def _():
            fetch_b(wid + 2 * _NW)

        writeout(wid + _NW)

        @pl.when(has_third)
        def _():
            wait_a()
            lax.fori_loop(0, _NGRP, pass_a, 0)
            wait_b()
            lax.fori_loop(0, _NGRP, pass_b, 0)
            writeout(wid + 2 * _NW)

    return gather_kernel


_gather = _make_gather()


def kernel(idx, appearance):
    out_t = _gather(idx.astype(jnp.int32), appearance.T)
    return out_t.T


# speedup vs baseline: 1.1542x; 1.1542x over previous
import functools

import jax
import jax.numpy as jnp
from jax import lax
from jax.experimental import pallas as pl
from jax.experimental.pallas import tpu as pltpu
from jax.experimental.pallas import tpu_sc as plsc

NUM_SIGNALS = 100000
ROW_WIDTH = 70
BATCH = 4096

_info = plsc.get_sparse_core_info()
_NC, _NS = _info.num_cores, _info.num_subcores
_NW = _NC * _NS
_UNROLL = 4
_NGRP = BATCH // (16 * _UNROLL)


def _make_gather():
    mesh = plsc.VectorSubcoreMesh(core_axis_name="c", subcore_axis_name="s")

    @functools.partial(
        pl.kernel,
        mesh=mesh,
        out_type=jax.ShapeDtypeStruct((ROW_WIDTH, BATCH), jnp.float32),
        scratch_types=[
            pltpu.VMEM((BATCH,), jnp.int32),
            pltpu.VMEM((NUM_SIGNALS,), jnp.float32),
            pltpu.VMEM((BATCH,), jnp.float32),
        ],
        compiler_params=pltpu.CompilerParams(
            use_tc_tiling_on_sc=True, needs_layout_passes=False
        ),
    )
    def gather_kernel(idx_hbm, table_hbm, out_hbm, idx_v, plane_v, res_v):
        wid = lax.axis_index("s") * _NC + lax.axis_index("c")
        pltpu.sync_copy(idx_hbm, idx_v)

        def do_plane(j):
            pltpu.sync_copy(table_hbm.at[j], plane_v)

            def gather_grp(k, carry):
                for u in range(_UNROLL):
                    o = (k * _UNROLL + u) * 16
                    g = plsc.load_gather(plane_v, [idx_v[pl.ds(o, 16)]])
                    res_v[pl.ds(o, 16)] = g
                return carry

            lax.fori_loop(0, _NGRP, gather_grp, 0)
            pltpu.sync_copy(res_v, out_hbm.at[j])

        do_plane(wid)
        do_plane(wid + _NW)

        @pl.when(wid < ROW_WIDTH - 2 * _NW)
        def _():
            do_plane(wid + 2 * _NW)

    return gather_kernel


_gather = _make_gather()


def kernel(idx, appearance):
    out_t = _gather(idx.astype(jnp.int32), appearance.T)
    return out_t.T


# speedup vs baseline: 1.1569x; 1.0024x over previous
import functools

import jax
import jax.numpy as jnp
from jax import lax
from jax.experimental import pallas as pl
from jax.experimental.pallas import tpu as pltpu
from jax.experimental.pallas import tpu_sc as plsc

NUM_SIGNALS = 100000
ROW_WIDTH = 70
BATCH = 4096

_info = plsc.get_sparse_core_info()
_NC, _NS = _info.num_cores, _info.num_subcores
_NW = _NC * _NS
_UNROLL = 4
_NGRP = BATCH // (16 * _UNROLL)


def _make_gather():
    mesh = plsc.VectorSubcoreMesh(core_axis_name="c", subcore_axis_name="s")

    @functools.partial(
        pl.kernel,
        mesh=mesh,
        out_type=jax.ShapeDtypeStruct((ROW_WIDTH, BATCH), jnp.float32),
        scratch_types=[
            pltpu.VMEM((BATCH,), jnp.int32),
            pltpu.VMEM((NUM_SIGNALS,), jnp.float32),
            pltpu.VMEM((BATCH,), jnp.float32),
        ],
        compiler_params=pltpu.CompilerParams(
            use_tc_tiling_on_sc=True,
            needs_layout_passes=False,
            skip_device_barrier=True,
        ),
    )
    def gather_kernel(idx_hbm, table_hbm, out_hbm, idx_v, plane_v, res_v):
        wid = lax.axis_index("s") * _NC + lax.axis_index("c")
        pltpu.sync_copy(idx_hbm, idx_v)

        def do_plane(j):
            pltpu.sync_copy(table_hbm.at[j], plane_v)

            def gather_grp(k, carry):
                for u in range(_UNROLL):
                    o = (k * _UNROLL + u) * 16
                    g = plsc.load_gather(plane_v, [idx_v[pl.ds(o, 16)]])
                    res_v[pl.ds(o, 16)] = g
                return carry

            lax.fori_loop(0, _NGRP, gather_grp, 0)
            pltpu.sync_copy(res_v, out_hbm.at[j])

        do_plane(wid)
        do_plane(wid + _NW)

        @pl.when(wid < ROW_WIDTH - 2 * _NW)
        def _():
            do_plane(wid + 2 * _NW)

    return gather_kernel


_gather = _make_gather()


def kernel(idx, appearance):
    out_t = _gather(idx.astype(jnp.int32), appearance.T)
    return out_t.T
